# baseline (speedup 1.0000x reference)
import jax
import jax.numpy as jnp
import numpy as np
from jax.experimental import pallas as pl
from jax.experimental.pallas import tpu as pltpu

RADIUS = 0.05
HMIN = -0.02
HMAX = 0.04
NSAMPLE = 32
EPS = 1e-5

TT = 64
RT = TT * NSAMPLE
CNT = np.float32(4 * 1024 * NSAMPLE)
_PARALLEL = pltpu.CompilerParams(dimension_semantics=("parallel", "parallel"))


def _bf(x):
    return x.astype(jnp.bfloat16).astype(jnp.float32)


def _fused_kernel(xyzt_ref, xyzfull_ref, ctr_ref, rot_ref, tri_ref, featt_ref,
                  w1f_ref, w1x_ref, y1_ref, ps1_ref, tabf_ref, tabx_ref):
    t = pl.program_id(1)
    n = featt_ref.shape[1]

    @pl.when(t == 0)
    def _():
        feat = featt_ref[0].astype(jnp.bfloat16)
        w1f = w1f_ref[...].astype(jnp.bfloat16)
        tabf_ref[...] = jnp.dot(feat, w1f, preferred_element_type=jnp.float32
                                ).astype(jnp.bfloat16)
        xyz_b = xyzfull_ref[0]
        hi = xyz_b.astype(jnp.bfloat16).astype(jnp.float32)
        lo = xyz_b - hi
        tabx_ref[...] = jnp.concatenate(
            [hi, lo, jnp.zeros((n, 2), jnp.float32)], axis=1
        ).astype(jnp.bfloat16)

    xt = xyzt_ref[0]
    c = ctr_ref[0]
    r = rot_ref[0]
    rel = [_bf(xt[k:k + 1, :] - c[:, k:k + 1]) for k in range(3)]
    rbf = [_bf(r[:, k:k + 1]) for k in range(9)]
    rc = []
    for cc in range(3):
        s = rel[0] * rbf[cc]
        s = s + rel[1] * rbf[3 + cc]
        s = s + rel[2] * rbf[6 + cc]
        rc.append(s)
    x_rot, y_rot, z_rot = rc
    d2 = y_rot * y_rot + z_rot * z_rot
    mask = (d2 < r[:, 2:3]) & (x_rot > HMIN) & (x_rot < HMAX)
    mb = jnp.where(mask, 1.0, 0.0).astype(jnp.bfloat16)
    csum = jnp.dot(mb, tri_ref[...], preferred_element_type=jnp.float32)
    cm = jnp.where(mask, csum, 0.0)
    tot = csum[:, n - 1:n]

    val3 = (jax.lax.broadcasted_iota(jnp.int32, (NSAMPLE, 1, 1), 0)
            + 1).astype(jnp.float32)
    cm3 = jnp.broadcast_to(cm.reshape(1, TT, n), (NSAMPLE, TT, n))
    oh = jnp.where(cm3 == val3, 1.0, 0.0).astype(jnp.bfloat16).reshape(RT, n)
    gf = jnp.dot(oh, tabf_ref[...], preferred_element_type=jnp.float32)
    x8 = jnp.dot(oh, tabx_ref[...], preferred_element_type=jnp.float32)

    filled = jnp.broadcast_to(
        (jnp.broadcast_to(tot.reshape(1, TT, 1), (NSAMPLE, TT, 1))
         >= val3), (NSAMPLE, TT, 1)).reshape(RT, 1)
    gf = jnp.where(filled, gf, tabf_ref[0:1, :].astype(jnp.float32))
    x8 = jnp.where(filled, x8, tabx_ref[0:1, :].astype(jnp.float32))

    crep = jnp.broadcast_to(c.reshape(1, TT, 3), (NSAMPLE, TT, 3)).reshape(RT, 3)
    rrep = jnp.broadcast_to(r.reshape(1, TT, 9), (NSAMPLE, TT, 9)).reshape(RT, 9)
    gk = [_bf((x8[:, k:k + 1] + x8[:, 3 + k:4 + k] - crep[:, k:k + 1])
              / np.float32(RADIUS)) for k in range(3)]
    rg = []
    for cc in range(3):
        acc = gk[0] * _bf(rrep[:, cc:cc + 1])
        acc = acc + gk[1] * _bf(rrep[:, 3 + cc:4 + cc])
        acc = acc + gk[2] * _bf(rrep[:, 6 + cc:7 + cc])
        rg.append(acc.astype(jnp.bfloat16))
    rg8 = jnp.concatenate(rg + [jnp.zeros((RT, 5), jnp.bfloat16)], axis=1)
    y1x = jnp.dot(rg8, w1x_ref[...].astype(jnp.bfloat16),
                  preferred_element_type=jnp.float32)
    y1 = gf + y1x
    ps1_ref[0, 0] = jnp.concatenate(
        [jnp.sum(y1, axis=0, keepdims=True),
         jnp.sum(y1 * y1, axis=0, keepdims=True)], axis=0)
    y1_ref[0] = y1.astype(jnp.bfloat16)


def _layer2_kernel(y1_ref, ps1_ref, w2t_ref, g1_ref, b1_ref,
                   mx_ref, mn_ref, ps2_ref):
    pst = jnp.sum(ps1_ref[...], axis=(0, 1))
    m = pst[0:1, :] / CNT
    var = pst[1:2, :] / CNT - m * m
    recip = 1.0 / jnp.sqrt(var + EPS)
    scale = recip * g1_ref[...]
    shift = b1_ref[...] - m * scale
    w2t = w2t_ref[...].astype(jnp.bfloat16)
    ssum = jnp.zeros((TT, 256), jnp.float32)
    ssq = jnp.zeros((TT, 256), jnp.float32)
    mx = None
    mn = None
    for s_ in range(NSAMPLE):
        y1s = y1_ref[0, 0, s_].astype(jnp.float32)
        h = jnp.maximum(y1s * scale + shift, 0.0)
        y2s = jnp.dot(h.astype(jnp.bfloat16), w2t,
                      preferred_element_type=jnp.float32)
        ssum = ssum + y2s
        ssq = ssq + y2s * y2s
        mx = y2s if mx is None else jnp.maximum(mx, y2s)
        mn = y2s if mn is None else jnp.minimum(mn, y2s)
    mx_ref[0] = mx
    mn_ref[0] = mn
    ps2_ref[0, 0] = jnp.concatenate(
        [jnp.sum(ssum, axis=0, keepdims=True),
         jnp.sum(ssq, axis=0, keepdims=True)], axis=0)


def _pool_kernel(mx_ref, mn_ref, ps2_ref, g2_ref, b2_ref, out_ref):
    pst = jnp.sum(ps2_ref[...], axis=(0, 1))
    m = pst[0:1, :] / CNT
    var = pst[1:2, :] / CNT - m * m
    recip = 1.0 / jnp.sqrt(var + EPS)
    scale = recip * g2_ref[...]
    shift = b2_ref[...] - m * scale
    sel = jnp.where(scale > 0.0, mx_ref[0], mn_ref[0])
    out_ref[0] = jnp.maximum(sel * scale + shift, 0.0)


def kernel(seed_xyz_graspable, seed_features_graspable, vp_rot, W1, g1, b1, W2, g2, b2):
    B, N, _ = seed_xyz_graspable.shape
    C = seed_features_graspable.shape[1]
    NT = N // TT
    xyz = seed_xyz_graspable
    xyzt = jnp.transpose(xyz, (0, 2, 1))
    rot9 = vp_rot.reshape(B, N, 9)
    featt = jnp.transpose(seed_features_graspable, (0, 2, 1))
    w1x = jnp.concatenate([W1[:, :3].T, jnp.zeros((5, 256), W1.dtype)], axis=0)
    w1f = W1[:, 3:].T
    w2t = W2.T
    jrow = jax.lax.broadcasted_iota(jnp.int32, (N, N), 0)
    jcol = jax.lax.broadcasted_iota(jnp.int32, (N, N), 1)
    tri = jnp.where(jrow <= jcol, 1.0, 0.0).astype(jnp.bfloat16)
    g1r, b1r = g1.reshape(1, 256), b1.reshape(1, 256)
    g2r, b2r = g2.reshape(1, 256), b2.reshape(1, 256)

    y1, ps1 = pl.pallas_call(
        _fused_kernel,
        grid=(B, NT),
        in_specs=[
            pl.BlockSpec((1, 3, N), lambda b, t: (b, 0, 0)),
            pl.BlockSpec((1, N, 3), lambda b, t: (b, 0, 0)),
            pl.BlockSpec((1, TT, 3), lambda b, t: (b, t, 0)),
            pl.BlockSpec((1, TT, 9), lambda b, t: (b, t, 0)),
            pl.BlockSpec((N, N), lambda b, t: (0, 0)),
            pl.BlockSpec((1, N, C), lambda b, t: (b, 0, 0)),
            pl.BlockSpec((C, 256), lambda b, t: (0, 0)),
            pl.BlockSpec((8, 256), lambda b, t: (0, 0)),
        ],
        out_specs=[
            pl.BlockSpec((1, NSAMPLE * TT, 256), lambda b, t: (b, t, 0)),
            pl.BlockSpec((1, 1, 2, 256), lambda b, t: (b, t, 0, 0)),
        ],
        out_shape=[
            jax.ShapeDtypeStruct((B, NSAMPLE * N, 256), jnp.bfloat16),
            jax.ShapeDtypeStruct((B, NT, 2, 256), jnp.float32),
        ],
        scratch_shapes=[
            pltpu.VMEM((N, 256), jnp.bfloat16),
            pltpu.VMEM((N, 8), jnp.bfloat16),
        ],
        compiler_params=_PARALLEL,
    )(xyzt, xyz, xyz, rot9, tri, featt, w1f, w1x)

    y1r = y1.reshape(B, N // TT, NSAMPLE, TT, 256)

    mx, mn, ps2 = pl.pallas_call(
        _layer2_kernel,
        grid=(B, NT),
        in_specs=[
            pl.BlockSpec((1, 1, NSAMPLE, TT, 256), lambda b, t: (b, t, 0, 0, 0)),
            pl.BlockSpec((B, NT, 2, 256), lambda b, t: (0, 0, 0, 0)),
            pl.BlockSpec((256, 256), lambda b, t: (0, 0)),
            pl.BlockSpec((1, 256), lambda b, t: (0, 0)),
            pl.BlockSpec((1, 256), lambda b, t: (0, 0)),
        ],
        out_specs=[
            pl.BlockSpec((1, TT, 256), lambda b, t: (b, t, 0)),
            pl.BlockSpec((1, TT, 256), lambda b, t: (b, t, 0)),
            pl.BlockSpec((1, 1, 2, 256), lambda b, t: (b, t, 0, 0)),
        ],
        out_shape=[
            jax.ShapeDtypeStruct((B, N, 256), jnp.float32),
            jax.ShapeDtypeStruct((B, N, 256), jnp.float32),
            jax.ShapeDtypeStruct((B, NT, 2, 256), jnp.float32),
        ],
        compiler_params=_PARALLEL,
    )(y1r, ps1, w2t, g1r, b1r)

    outp = pl.pallas_call(
        _pool_kernel,
        grid=(B, NT),
        in_specs=[
            pl.BlockSpec((1, TT, 256), lambda b, t: (b, t, 0)),
            pl.BlockSpec((1, TT, 256), lambda b, t: (b, t, 0)),
            pl.BlockSpec((B, NT, 2, 256), lambda b, t: (0, 0, 0, 0)),
            pl.BlockSpec((1, 256), lambda b, t: (0, 0)),
            pl.BlockSpec((1, 256), lambda b, t: (0, 0)),
        ],
        out_specs=pl.BlockSpec((1, TT, 256), lambda b, t: (b, t, 0)),
        out_shape=jax.ShapeDtypeStruct((B, N, 256), jnp.float32),
        compiler_params=_PARALLEL,
    )(mx, mn, ps2, g2r, b2r)

    return jnp.transpose(outp, (0, 2, 1))


# speedup vs baseline: 1.3703x; 1.3703x over previous
import jax
import jax.numpy as jnp
import numpy as np
from jax.experimental import pallas as pl
from jax.experimental.pallas import tpu as pltpu

RADIUS = 0.05
HMIN = -0.02
HMAX = 0.04
NSAMPLE = 32
EPS = 1e-5

TT = 64
CNT = np.float32(4 * 1024 * NSAMPLE)


def _bf(x):
    return x.astype(jnp.bfloat16).astype(jnp.float32)


def _fused_kernel(xyzt_ref, xyzfull_ref, ctr_ref, rot_ref, tri_ref, featt_ref,
                  w1f_ref, w1x_ref, y1_ref, st1_ref, tab_ref, acc_ref):
    b = pl.program_id(0)
    t = pl.program_id(1)
    nb = pl.num_programs(0)
    nt = pl.num_programs(1)
    n = featt_ref.shape[1]

    @pl.when(jnp.logical_and(b == 0, t == 0))
    def _():
        acc_ref[...] = jnp.zeros_like(acc_ref)

    @pl.when(t == 0)
    def _():
        feat = featt_ref[0].astype(jnp.bfloat16)
        w1f = w1f_ref[...].astype(jnp.bfloat16)
        f1 = jnp.dot(feat, w1f, preferred_element_type=jnp.float32)
        xyz_b = xyzfull_ref[0] / np.float32(RADIUS)
        hi = xyz_b.astype(jnp.bfloat16).astype(jnp.float32)
        lo = xyz_b - hi
        tab_ref[...] = jnp.concatenate(
            [f1, hi, lo, jnp.zeros((n, 2), jnp.float32)], axis=1
        ).astype(jnp.bfloat16)

    xt = xyzt_ref[0]
    c = ctr_ref[0]
    r = rot_ref[0]
    rel = [_bf(xt[k:k + 1, :] - c[:, k:k + 1]) for k in range(3)]
    rbf = [_bf(r[:, k:k + 1]) for k in range(9)]
    rc = []
    for cc in range(3):
        s = rel[0] * rbf[cc]
        s = s + rel[1] * rbf[3 + cc]
        s = s + rel[2] * rbf[6 + cc]
        rc.append(s)
    x_rot, y_rot, z_rot = rc
    d2 = y_rot * y_rot + z_rot * z_rot
    mask = (d2 < r[:, 2:3]) & (x_rot > HMIN) & (x_rot < HMAX)
    mb = jnp.where(mask, 1.0, 0.0).astype(jnp.bfloat16)
    csum = jnp.dot(mb, tri_ref[...], preferred_element_type=jnp.float32)
    cm = jnp.where(mask, csum, 0.0)
    tot = csum[:, n - 1:n]

    w1x = w1x_ref[...]
    mk = []
    for k in range(3):
        acc = rbf[3 * k] * _bf(w1x[0:1, :])
        acc = acc + rbf[3 * k + 1] * _bf(w1x[1:2, :])
        acc = acc + rbf[3 * k + 2] * _bf(w1x[2:3, :])
        mk.append(acc)

    tab = tab_ref[...]
    tab0 = tab[0:1, :].astype(jnp.float32)
    cdiv = c / np.float32(RADIUS)

    ssum = jnp.zeros((TT, 256), jnp.float32)
    ssq = jnp.zeros((TT, 256), jnp.float32)
    GRP = 4
    for g_ in range(NSAMPLE // GRP):
        ohg = jnp.concatenate(
            [jnp.where(cm == np.float32(g_ * GRP + j + 1), 1.0, 0.0
                       ).astype(jnp.bfloat16) for j in range(GRP)], axis=0)
        res = jnp.dot(ohg, tab, preferred_element_type=jnp.float32)
        for j in range(GRP):
            s_ = g_ * GRP + j
            filled = tot > np.float32(s_)
            rj = res[j * TT:(j + 1) * TT, :]
            rj = jnp.where(filled, rj, tab0)
            y1s = rj[:, 0:256]
            for k in range(3):
                gk = _bf(rj[:, 256 + k:257 + k] + rj[:, 259 + k:260 + k]
                         - cdiv[:, k:k + 1])
                y1s = y1s + gk * mk[k]
            ssum = ssum + y1s
            ssq = ssq + y1s * y1s
            y1_ref[0, s_] = y1s.astype(jnp.bfloat16)
    acc_ref[0:1, :] = acc_ref[0:1, :] + jnp.sum(ssum, axis=0, keepdims=True)
    acc_ref[1:2, :] = acc_ref[1:2, :] + jnp.sum(ssq, axis=0, keepdims=True)

    @pl.when(jnp.logical_and(b == nb - 1, t == nt - 1))
    def _():
        st1_ref[...] = acc_ref[...]


def _layer2_kernel(y1_ref, st1_ref, w2t_ref, g1_ref, b1_ref,
                   mx_ref, mn_ref, st2_ref, acc_ref):
    b = pl.program_id(0)
    t = pl.program_id(1)
    nb = pl.num_programs(0)
    nt = pl.num_programs(1)

    @pl.when(jnp.logical_and(b == 0, t == 0))
    def _():
        acc_ref[...] = jnp.zeros_like(acc_ref)

    m = st1_ref[0:1, :] / CNT
    var = st1_ref[1:2, :] / CNT - m * m
    recip = 1.0 / jnp.sqrt(var + EPS)
    scale = recip * g1_ref[...]
    shift = b1_ref[...] - m * scale
    w2t = w2t_ref[...].astype(jnp.bfloat16)
    ssum = jnp.zeros((TT, 256), jnp.float32)
    ssq = jnp.zeros((TT, 256), jnp.float32)
    mx = None
    mn = None
    GRP = 4
    for g_ in range(NSAMPLE // GRP):
        hg = jnp.concatenate(
            [jnp.maximum(y1_ref[0, g_ * GRP + j].astype(jnp.float32) * scale
                         + shift, 0.0).astype(jnp.bfloat16)
             for j in range(GRP)], axis=0)
        y2g = jnp.dot(hg, w2t, preferred_element_type=jnp.float32)
        for j in range(GRP):
            y2s = y2g[j * TT:(j + 1) * TT, :]
            ssum = ssum + y2s
            ssq = ssq + y2s * y2s
            mx = y2s if mx is None else jnp.maximum(mx, y2s)
            mn = y2s if mn is None else jnp.minimum(mn, y2s)
    mx_ref[0] = mx
    mn_ref[0] = mn
    acc_ref[0:1, :] = acc_ref[0:1, :] + jnp.sum(ssum, axis=0, keepdims=True)
    acc_ref[1:2, :] = acc_ref[1:2, :] + jnp.sum(ssq, axis=0, keepdims=True)

    @pl.when(jnp.logical_and(b == nb - 1, t == nt - 1))
    def _():
        st2_ref[...] = acc_ref[...]


def _pool_kernel(mx_ref, mn_ref, st2_ref, g2_ref, b2_ref, out_ref):
    m = st2_ref[0:1, :] / CNT
    var = st2_ref[1:2, :] / CNT - m * m
    recip = 1.0 / jnp.sqrt(var + EPS)
    scale = recip * g2_ref[...]
    shift = b2_ref[...] - m * scale
    sel = jnp.where(scale > 0.0, mx_ref[0], mn_ref[0])
    out_ref[0] = jnp.maximum(sel * scale + shift, 0.0)


def kernel(seed_xyz_graspable, seed_features_graspable, vp_rot, W1, g1, b1, W2, g2, b2):
    B, N, _ = seed_xyz_graspable.shape
    C = seed_features_graspable.shape[1]
    xyz = seed_xyz_graspable
    xyzt = jnp.transpose(xyz, (0, 2, 1))
    rot9 = vp_rot.reshape(B, N, 9)
    featt = jnp.transpose(seed_features_graspable, (0, 2, 1))
    w1x = jnp.concatenate([W1[:, :3].T, jnp.zeros((5, 256), W1.dtype)], axis=0)
    w1f = W1[:, 3:].T
    w2t = W2.T
    jrow = jax.lax.broadcasted_iota(jnp.int32, (N, N), 0)
    jcol = jax.lax.broadcasted_iota(jnp.int32, (N, N), 1)
    tri = jnp.where(jrow <= jcol, 1.0, 0.0).astype(jnp.bfloat16)
    g1r, b1r = g1.reshape(1, 256), b1.reshape(1, 256)
    g2r, b2r = g2.reshape(1, 256), b2.reshape(1, 256)

    y1, st1 = pl.pallas_call(
        _fused_kernel,
        grid=(B, N // TT),
        in_specs=[
            pl.BlockSpec((1, 3, N), lambda b, t: (b, 0, 0)),
            pl.BlockSpec((1, N, 3), lambda b, t: (b, 0, 0)),
            pl.BlockSpec((1, TT, 3), lambda b, t: (b, t, 0)),
            pl.BlockSpec((1, TT, 9), lambda b, t: (b, t, 0)),
            pl.BlockSpec((N, N), lambda b, t: (0, 0)),
            pl.BlockSpec((1, N, C), lambda b, t: (b, 0, 0)),
            pl.BlockSpec((C, 256), lambda b, t: (0, 0)),
            pl.BlockSpec((8, 256), lambda b, t: (0, 0)),
        ],
        out_specs=[
            pl.BlockSpec((1, NSAMPLE, TT, 256), lambda b, t: (b, 0, t, 0)),
            pl.BlockSpec((8, 256), lambda b, t: (0, 0)),
        ],
        out_shape=[
            jax.ShapeDtypeStruct((B, NSAMPLE, N, 256), jnp.bfloat16),
            jax.ShapeDtypeStruct((8, 256), jnp.float32),
        ],
        scratch_shapes=[
            pltpu.VMEM((N, 264), jnp.bfloat16),
            pltpu.VMEM((8, 256), jnp.float32),
        ],
    )(xyzt, xyz, xyz, rot9, tri, featt, w1f, w1x)

    mx, mn, st2 = pl.pallas_call(
        _layer2_kernel,
        grid=(B, N // TT),
        in_specs=[
            pl.BlockSpec((1, NSAMPLE, TT, 256), lambda b, t: (b, 0, t, 0)),
            pl.BlockSpec((8, 256), lambda b, t: (0, 0)),
            pl.BlockSpec((256, 256), lambda b, t: (0, 0)),
            pl.BlockSpec((1, 256), lambda b, t: (0, 0)),
            pl.BlockSpec((1, 256), lambda b, t: (0, 0)),
        ],
        out_specs=[
            pl.BlockSpec((1, TT, 256), lambda b, t: (b, t, 0)),
            pl.BlockSpec((1, TT, 256), lambda b, t: (b, t, 0)),
            pl.BlockSpec((8, 256), lambda b, t: (0, 0)),
        ],
        out_shape=[
            jax.ShapeDtypeStruct((B, N, 256), jnp.float32),
            jax.ShapeDtypeStruct((B, N, 256), jnp.float32),
            jax.ShapeDtypeStruct((8, 256), jnp.float32),
        ],
        scratch_shapes=[pltpu.VMEM((8, 256), jnp.float32)],
    )(y1, st1, w2t, g1r, b1r)

    outp = pl.pallas_call(
        _pool_kernel,
        grid=(B, N // TT),
        in_specs=[
            pl.BlockSpec((1, TT, 256), lambda b, t: (b, t, 0)),
            pl.BlockSpec((1, TT, 256), lambda b, t: (b, t, 0)),
            pl.BlockSpec((8, 256), lambda b, t: (0, 0)),
            pl.BlockSpec((1, 256), lambda b, t: (0, 0)),
            pl.BlockSpec((1, 256), lambda b, t: (0, 0)),
        ],
        out_specs=pl.BlockSpec((1, TT, 256), lambda b, t: (b, t, 0)),
        out_shape=jax.ShapeDtypeStruct((B, N, 256), jnp.float32),
    )(mx, mn, st2, g2r, b2r)

    return jnp.transpose(outp, (0, 2, 1))


# speedup vs baseline: 1.4894x; 1.0869x over previous
import jax
import jax.numpy as jnp
import numpy as np
from jax.experimental import pallas as pl
from jax.experimental.pallas import tpu as pltpu

RADIUS = 0.05
HMIN = -0.02
HMAX = 0.04
NSAMPLE = 32
EPS = 1e-5

TT = 128
CNT = np.float32(4 * 1024 * NSAMPLE)


def _bf(x):
    return x.astype(jnp.bfloat16).astype(jnp.float32)


def _fused_kernel(xyzt_ref, xyzfull_ref, ctr_ref, rot_ref, tri_ref, featt_ref,
                  w1f_ref, w1x_ref, y1_ref, st1_ref, tab_ref, acc_ref):
    b = pl.program_id(0)
    t = pl.program_id(1)
    nb = pl.num_programs(0)
    nt = pl.num_programs(1)
    n = featt_ref.shape[1]

    @pl.when(jnp.logical_and(b == 0, t == 0))
    def _():
        acc_ref[...] = jnp.zeros_like(acc_ref)

    @pl.when(t == 0)
    def _():
        feat = featt_ref[0].astype(jnp.bfloat16)
        w1f = w1f_ref[...].astype(jnp.bfloat16)
        f1 = jnp.dot(feat, w1f, preferred_element_type=jnp.float32)
        xyz_b = xyzfull_ref[0] / np.float32(RADIUS)
        hi = xyz_b.astype(jnp.bfloat16).astype(jnp.float32)
        lo = xyz_b - hi
        tab_ref[...] = jnp.concatenate(
            [f1, hi, lo, jnp.zeros((n, 2), jnp.float32)], axis=1
        ).astype(jnp.bfloat16)

    xt = xyzt_ref[0]
    c = ctr_ref[0]
    r = rot_ref[0]
    rel = [_bf(xt[k:k + 1, :] - c[:, k:k + 1]) for k in range(3)]
    rbf = [_bf(r[:, k:k + 1]) for k in range(9)]
    rc = []
    for cc in range(3):
        s = rel[0] * rbf[cc]
        s = s + rel[1] * rbf[3 + cc]
        s = s + rel[2] * rbf[6 + cc]
        rc.append(s)
    x_rot, y_rot, z_rot = rc
    d2 = y_rot * y_rot + z_rot * z_rot
    mask = (d2 < r[:, 2:3]) & (x_rot > HMIN) & (x_rot < HMAX)
    mb = jnp.where(mask, 1.0, 0.0).astype(jnp.bfloat16)
    csum = jnp.dot(mb, tri_ref[...], preferred_element_type=jnp.float32)
    cm = jnp.where(mask, csum, 0.0)
    tot = csum[:, n - 1:n]

    w1x = w1x_ref[...]
    mk = []
    for k in range(3):
        acc = rbf[3 * k] * _bf(w1x[0:1, :])
        acc = acc + rbf[3 * k + 1] * _bf(w1x[1:2, :])
        acc = acc + rbf[3 * k + 2] * _bf(w1x[2:3, :])
        mk.append(acc)

    tab = tab_ref[...]
    tab0 = tab[0:1, :].astype(jnp.float32)
    cdiv = c / np.float32(RADIUS)

    ssum = jnp.zeros((TT, 256), jnp.float32)
    ssq = jnp.zeros((TT, 256), jnp.float32)
    GRP = 8
    for g_ in range(NSAMPLE // GRP):
        ohg = jnp.concatenate(
            [jnp.where(cm == np.float32(g_ * GRP + j + 1), 1.0, 0.0
                       ).astype(jnp.bfloat16) for j in range(GRP)], axis=0)
        res = jnp.dot(ohg, tab, preferred_element_type=jnp.float32)
        for j in range(GRP):
            s_ = g_ * GRP + j
            filled = tot > np.float32(s_)
            rj = res[j * TT:(j + 1) * TT, :]
            rj = jnp.where(filled, rj, tab0)
            y1s = rj[:, 0:256]
            for k in range(3):
                gk = _bf(rj[:, 256 + k:257 + k] + rj[:, 259 + k:260 + k]
                         - cdiv[:, k:k + 1])
                y1s = y1s + gk * mk[k]
            ssum = ssum + y1s
            ssq = ssq + y1s * y1s
            y1_ref[0, s_] = y1s.astype(jnp.bfloat16)
    acc_ref[0:1, :] = acc_ref[0:1, :] + jnp.sum(ssum, axis=0, keepdims=True)
    acc_ref[1:2, :] = acc_ref[1:2, :] + jnp.sum(ssq, axis=0, keepdims=True)

    @pl.when(jnp.logical_and(b == nb - 1, t == nt - 1))
    def _():
        st1_ref[...] = acc_ref[...]


def _layer2_kernel(y1_ref, st1_ref, w2t_ref, g1_ref, b1_ref,
                   mx_ref, mn_ref, st2_ref, acc_ref):
    b = pl.program_id(0)
    t = pl.program_id(1)
    nb = pl.num_programs(0)
    nt = pl.num_programs(1)

    @pl.when(jnp.logical_and(b == 0, t == 0))
    def _():
        acc_ref[...] = jnp.zeros_like(acc_ref)

    m = st1_ref[0:1, :] / CNT
    var = st1_ref[1:2, :] / CNT - m * m
    recip = 1.0 / jnp.sqrt(var + EPS)
    scale = recip * g1_ref[...]
    shift = b1_ref[...] - m * scale
    w2t = w2t_ref[...].astype(jnp.bfloat16)
    ssum = jnp.zeros((TT, 256), jnp.float32)
    ssq = jnp.zeros((TT, 256), jnp.float32)
    mx = None
    mn = None
    GRP = 8
    for g_ in range(NSAMPLE // GRP):
        hg = jnp.concatenate(
            [jnp.maximum(y1_ref[0, g_ * GRP + j].astype(jnp.float32) * scale
                         + shift, 0.0).astype(jnp.bfloat16)
             for j in range(GRP)], axis=0)
        y2g = jnp.dot(hg, w2t, preferred_element_type=jnp.float32)
        for j in range(GRP):
            y2s = y2g[j * TT:(j + 1) * TT, :]
            ssum = ssum + y2s
            ssq = ssq + y2s * y2s
            mx = y2s if mx is None else jnp.maximum(mx, y2s)
            mn = y2s if mn is None else jnp.minimum(mn, y2s)
    mx_ref[0] = mx
    mn_ref[0] = mn
    acc_ref[0:1, :] = acc_ref[0:1, :] + jnp.sum(ssum, axis=0, keepdims=True)
    acc_ref[1:2, :] = acc_ref[1:2, :] + jnp.sum(ssq, axis=0, keepdims=True)

    @pl.when(jnp.logical_and(b == nb - 1, t == nt - 1))
    def _():
        st2_ref[...] = acc_ref[...]


def _pool_kernel(mx_ref, mn_ref, st2_ref, g2_ref, b2_ref, out_ref):
    m = st2_ref[0:1, :] / CNT
    var = st2_ref[1:2, :] / CNT - m * m
    recip = 1.0 / jnp.sqrt(var + EPS)
    scale = recip * g2_ref[...]
    shift = b2_ref[...] - m * scale
    sel = jnp.where(scale > 0.0, mx_ref[0], mn_ref[0])
    out_ref[0] = jnp.maximum(sel * scale + shift, 0.0)


def kernel(seed_xyz_graspable, seed_features_graspable, vp_rot, W1, g1, b1, W2, g2, b2):
    B, N, _ = seed_xyz_graspable.shape
    C = seed_features_graspable.shape[1]
    xyz = seed_xyz_graspable
    xyzt = jnp.transpose(xyz, (0, 2, 1))
    rot9 = vp_rot.reshape(B, N, 9)
    featt = jnp.transpose(seed_features_graspable, (0, 2, 1))
    w1x = jnp.concatenate([W1[:, :3].T, jnp.zeros((5, 256), W1.dtype)], axis=0)
    w1f = W1[:, 3:].T
    w2t = W2.T
    jrow = jax.lax.broadcasted_iota(jnp.int32, (N, N), 0)
    jcol = jax.lax.broadcasted_iota(jnp.int32, (N, N), 1)
    tri = jnp.where(jrow <= jcol, 1.0, 0.0).astype(jnp.bfloat16)
    g1r, b1r = g1.reshape(1, 256), b1.reshape(1, 256)
    g2r, b2r = g2.reshape(1, 256), b2.reshape(1, 256)

    y1, st1 = pl.pallas_call(
        _fused_kernel,
        grid=(B, N // TT),
        in_specs=[
            pl.BlockSpec((1, 3, N), lambda b, t: (b, 0, 0)),
            pl.BlockSpec((1, N, 3), lambda b, t: (b, 0, 0)),
            pl.BlockSpec((1, TT, 3), lambda b, t: (b, t, 0)),
            pl.BlockSpec((1, TT, 9), lambda b, t: (b, t, 0)),
            pl.BlockSpec((N, N), lambda b, t: (0, 0)),
            pl.BlockSpec((1, N, C), lambda b, t: (b, 0, 0)),
            pl.BlockSpec((C, 256), lambda b, t: (0, 0)),
            pl.BlockSpec((8, 256), lambda b, t: (0, 0)),
        ],
        out_specs=[
            pl.BlockSpec((1, NSAMPLE, TT, 256), lambda b, t: (b, 0, t, 0)),
            pl.BlockSpec((8, 256), lambda b, t: (0, 0)),
        ],
        out_shape=[
            jax.ShapeDtypeStruct((B, NSAMPLE, N, 256), jnp.bfloat16),
            jax.ShapeDtypeStruct((8, 256), jnp.float32),
        ],
        scratch_shapes=[
            pltpu.VMEM((N, 264), jnp.bfloat16),
            pltpu.VMEM((8, 256), jnp.float32),
        ],
    )(xyzt, xyz, xyz, rot9, tri, featt, w1f, w1x)

    mx, mn, st2 = pl.pallas_call(
        _layer2_kernel,
        grid=(B, N // TT),
        in_specs=[
            pl.BlockSpec((1, NSAMPLE, TT, 256), lambda b, t: (b, 0, t, 0)),
            pl.BlockSpec((8, 256), lambda b, t: (0, 0)),
            pl.BlockSpec((256, 256), lambda b, t: (0, 0)),
            pl.BlockSpec((1, 256), lambda b, t: (0, 0)),
            pl.BlockSpec((1, 256), lambda b, t: (0, 0)),
        ],
        out_specs=[
            pl.BlockSpec((1, TT, 256), lambda b, t: (b, t, 0)),
            pl.BlockSpec((1, TT, 256), lambda b, t: (b, t, 0)),
            pl.BlockSpec((8, 256), lambda b, t: (0, 0)),
        ],
        out_shape=[
            jax.ShapeDtypeStruct((B, N, 256), jnp.float32),
            jax.ShapeDtypeStruct((B, N, 256), jnp.float32),
            jax.ShapeDtypeStruct((8, 256), jnp.float32),
        ],
        scratch_shapes=[pltpu.VMEM((8, 256), jnp.float32)],
    )(y1, st1, w2t, g1r, b1r)

    outp = pl.pallas_call(
        _pool_kernel,
        grid=(B, N // TT),
        in_specs=[
            pl.BlockSpec((1, TT, 256), lambda b, t: (b, t, 0)),
            pl.BlockSpec((1, TT, 256), lambda b, t: (b, t, 0)),
            pl.BlockSpec((8, 256), lambda b, t: (0, 0)),
            pl.BlockSpec((1, 256), lambda b, t: (0, 0)),
            pl.BlockSpec((1, 256), lambda b, t: (0, 0)),
        ],
        out_specs=pl.BlockSpec((1, TT, 256), lambda b, t: (b, t, 0)),
        out_shape=jax.ShapeDtypeStruct((B, N, 256), jnp.float32),
    )(mx, mn, st2, g2r, b2r)

    return jnp.transpose(outp, (0, 2, 1))
